# baseline (speedup 1.0000x reference)
import functools

import jax
import jax.numpy as jnp
from jax import lax
from jax.experimental import pallas as pl
from jax.experimental.pallas import tpu as pltpu
from jax.experimental.pallas import tpu_sc as plsc

_EPS = 1e-5
_NC = 2
_NS = 16
_NW = _NC * _NS
_CH = 128
_GPB = 2
_BUF = _CH * _GPB
_NBUF = 2


def _ln_body(t_ref, g_ref, b_ref, o_ref):
    t = t_ref[...]
    mean = jnp.mean(t, axis=1, keepdims=True)
    var = jnp.mean((t - mean) ** 2, axis=1, keepdims=True)
    o_ref[...] = (t - mean) * lax.rsqrt(var + _EPS) * g_ref[...] + b_ref[...]


def _make_sc_gather(n, d):
    rw = n // _NW
    steps = rw // _BUF
    mesh = plsc.VectorSubcoreMesh(core_axis_name="c", subcore_axis_name="s")

    @functools.partial(
        pl.kernel,
        out_type=jax.ShapeDtypeStruct((n, d), jnp.float32),
        mesh=mesh,
        scratch_types=[
            pltpu.VMEM((rw,), jnp.int32),
            pltpu.VMEM((_NBUF, _BUF, d), jnp.float32),
            pltpu.SemaphoreType.DMA,
            pltpu.SemaphoreType.DMA,
        ],
    )
    def sc_gather(nt_hbm, idx_hbm, out_hbm, idx_v, rows_v, gsem, ssem):
        wid = lax.axis_index("s") * _NC + lax.axis_index("c")
        base = wid * rw
        pltpu.sync_copy(idx_hbm.at[pl.ds(base, rw)], idx_v)

        def fire_gathers(step, b):
            for j in range(_GPB):
                pltpu.async_copy(
                    nt_hbm.at[idx_v.at[pl.ds(step * _BUF + j * _CH, _CH)]],
                    rows_v.at[b, pl.ds(j * _CH, _CH)],
                    gsem,
                )

        def wait_gathers(b):
            for j in range(_GPB):
                pltpu.make_async_copy(
                    nt_hbm.at[idx_v.at[pl.ds(j * _CH, _CH)]],
                    rows_v.at[b, pl.ds(j * _CH, _CH)],
                    gsem,
                ).wait()

        def fire_store(step, b):
            pltpu.async_copy(
                rows_v.at[b], out_hbm.at[pl.ds(base + step * _BUF, _BUF)], ssem)

        def wait_store(step, b):
            pltpu.make_async_copy(
                rows_v.at[b], out_hbm.at[pl.ds(base + step * _BUF, _BUF)], ssem,
            ).wait()

        fire_gathers(0, 0)

        def body(g, carry):
            for b in range(_NBUF):
                step = g * _NBUF + b
                nxt = (b + 1) % _NBUF

                @pl.when(step + 1 < steps)
                def _():
                    @pl.when(step + 1 >= _NBUF)
                    def _():
                        wait_store(step + 1 - _NBUF, nxt)

                    fire_gathers(step + 1, nxt)

                wait_gathers(b)
                fire_store(step, b)
            return carry

        lax.fori_loop(0, steps // _NBUF, body, 0)
        wait_store(steps - 2, (steps - 2) % _NBUF)
        wait_store(steps - 1, (steps - 1) % _NBUF)

    return sc_gather


def kernel(grid, table, gamma, beta):
    batch, h, w = grid.shape
    v, d = table.shape
    n = batch * h * w

    tpad = jnp.zeros((16, d), jnp.float32).at[:v].set(table.astype(jnp.float32))
    nt = pl.pallas_call(
        _ln_body,
        out_shape=jax.ShapeDtypeStruct((16, d), jnp.float32),
    )(tpad, gamma.reshape(1, d), beta.reshape(1, d))

    idx = grid.reshape(n).astype(jnp.int32)
    out = _make_sc_gather(n, d)(nt, idx)
    return out.reshape(batch, h * w, d)


# speedup vs baseline: 1.0463x; 1.0463x over previous
import functools

import jax
import jax.numpy as jnp
from jax import lax
from jax.experimental import pallas as pl
from jax.experimental.pallas import tpu as pltpu
from jax.experimental.pallas import tpu_sc as plsc

_EPS = 1e-5
_NC = 2
_NS = 16
_NW = _NC * _NS
_L = 16
_D = 128
_BUF = 256
_NBUF = 2


def _ln_body(t_ref, g_ref, b_ref, o_ref):
    t = t_ref[...]
    mean = jnp.mean(t, axis=1, keepdims=True)
    var = jnp.mean((t - mean) ** 2, axis=1, keepdims=True)
    o_ref[...] = (t - mean) * lax.rsqrt(var + _EPS) * g_ref[...] + b_ref[...]


def _make_sc_gather(n):
    rw = n // _NW
    steps = rw // _BUF
    groups = _BUF // _L
    mesh = plsc.VectorSubcoreMesh(core_axis_name="c", subcore_axis_name="s")

    @functools.partial(
        pl.kernel,
        out_type=jax.ShapeDtypeStruct((n * _D,), jnp.float32),
        mesh=mesh,
        compiler_params=pltpu.CompilerParams(needs_layout_passes=False),
        scratch_types=[
            pltpu.VMEM((16 * _D,), jnp.float32),
            pltpu.VMEM((rw,), jnp.int32),
            pltpu.VMEM((_BUF * _D,), jnp.float32),
            pltpu.VMEM((_BUF * _D,), jnp.float32),
            pltpu.SemaphoreType.DMA,
        ],
    )
    def sc_gather(nt_hbm, idx_hbm, out_hbm, nt_v, idx_v, buf0_v, buf1_v, ssem):
        bufs = [buf0_v, buf1_v]
        wid = lax.axis_index("s") * _NC + lax.axis_index("c")
        base = wid * rw
        pltpu.sync_copy(nt_hbm, nt_v)
        pltpu.sync_copy(idx_hbm.at[pl.ds(base, rw)], idx_v)

        lane_off = lax.broadcasted_iota(jnp.int32, (_L,), 0) * _D

        def construct(step, b):
            def grp_body(g, c):
                idx16 = idx_v[pl.ds(step * _BUF + g * _L, _L)]
                gbase = lax.shift_left(idx16, 7)
                sbase = lane_off + g * (_L * _D)
                for dd in range(_D):
                    v = plsc.load_gather(nt_v, [gbase + dd])
                    plsc.store_scatter(bufs[b], [sbase + dd], v)
                return c

            lax.fori_loop(0, groups, grp_body, 0)

        def fire_store(step, b):
            pltpu.async_copy(
                bufs[b],
                out_hbm.at[pl.ds((base + step * _BUF) * _D, _BUF * _D)],
                ssem,
            )

        def wait_store(step, b):
            pltpu.make_async_copy(
                bufs[b],
                out_hbm.at[pl.ds((base + step * _BUF) * _D, _BUF * _D)],
                ssem,
            ).wait()

        def body(go, carry):
            for b in range(_NBUF):
                step = go * _NBUF + b

                @pl.when(step >= _NBUF)
                def _():
                    wait_store(step - _NBUF, b)

                construct(step, b)
                fire_store(step, b)
            return carry

        lax.fori_loop(0, steps // _NBUF, body, 0)
        wait_store(steps - 2, (steps - 2) % _NBUF)
        wait_store(steps - 1, (steps - 1) % _NBUF)

    return sc_gather


def kernel(grid, table, gamma, beta):
    batch, h, w = grid.shape
    v, d = table.shape
    n = batch * h * w

    tpad = jnp.zeros((16, d), jnp.float32).at[:v].set(table.astype(jnp.float32))
    nt = pl.pallas_call(
        _ln_body,
        out_shape=jax.ShapeDtypeStruct((16, d), jnp.float32),
    )(tpad, gamma.reshape(1, d), beta.reshape(1, d))

    idx = grid.reshape(n).astype(jnp.int32)
    out = _make_sc_gather(n)(nt.reshape(16 * d), idx)
    return out.reshape(batch, h * w, d)


# speedup vs baseline: 1.6425x; 1.5698x over previous
import functools

import jax
import jax.numpy as jnp
from jax import lax
from jax.experimental import pallas as pl
from jax.experimental.pallas import tpu as pltpu
from jax.experimental.pallas import tpu_sc as plsc

_EPS = 1e-5
_NC = 2
_NS = 16
_NW = _NC * _NS
_L = 16
_D = 128
_BUF = 256
_NBUF = 2


def _ln_body(t_ref, g_ref, b_ref, o_ref):
    t = t_ref[...]
    mean = jnp.mean(t, axis=1, keepdims=True)
    var = jnp.mean((t - mean) ** 2, axis=1, keepdims=True)
    o_ref[...] = (t - mean) * lax.rsqrt(var + _EPS) * g_ref[...] + b_ref[...]


def _make_sc_gather(n):
    rw = n // _NW
    steps = rw // _BUF
    groups = _BUF // _L
    mesh = plsc.VectorSubcoreMesh(core_axis_name="c", subcore_axis_name="s")

    @functools.partial(
        pl.kernel,
        out_type=jax.ShapeDtypeStruct((n * _D,), jnp.float32),
        mesh=mesh,
        compiler_params=pltpu.CompilerParams(needs_layout_passes=False),
        scratch_types=[
            pltpu.VMEM((16 * _D,), jnp.float32),
            pltpu.VMEM((rw,), jnp.int32),
            pltpu.VMEM((_BUF * _D,), jnp.float32),
            pltpu.VMEM((_BUF * _D,), jnp.float32),
            pltpu.SemaphoreType.DMA,
        ],
    )
    def sc_gather(nt_hbm, idx_hbm, out_hbm, nt_v, idx_v, buf0_v, buf1_v, ssem):
        bufs = [buf0_v, buf1_v]
        wid = lax.axis_index("s") * _NC + lax.axis_index("c")
        base = wid * rw
        pltpu.sync_copy(nt_hbm, nt_v)
        pltpu.sync_copy(idx_hbm.at[pl.ds(base, rw)], idx_v)

        lane_off = lax.broadcasted_iota(jnp.int32, (_L,), 0) * _D

        def construct(step, b):
            @plsc.parallel_loop(0, groups)
            def grp_body(g):
                idx16 = idx_v[pl.ds(step * _BUF + g * _L, _L)]
                gbase = lax.shift_left(idx16, 7)
                sbase = lane_off + g * (_L * _D)
                for dd in range(_D):
                    v = plsc.load_gather(nt_v, [gbase + dd])
                    plsc.store_scatter(bufs[b], [sbase + dd], v)

        def fire_store(step, b):
            pltpu.async_copy(
                bufs[b],
                out_hbm.at[pl.ds((base + step * _BUF) * _D, _BUF * _D)],
                ssem,
            )

        def wait_store(step, b):
            pltpu.make_async_copy(
                bufs[b],
                out_hbm.at[pl.ds((base + step * _BUF) * _D, _BUF * _D)],
                ssem,
            ).wait()

        def body(go, carry):
            for b in range(_NBUF):
                step = go * _NBUF + b

                @pl.when(step >= _NBUF)
                def _():
                    wait_store(step - _NBUF, b)

                construct(step, b)
                fire_store(step, b)
            return carry

        lax.fori_loop(0, steps // _NBUF, body, 0)
        wait_store(steps - 2, (steps - 2) % _NBUF)
        wait_store(steps - 1, (steps - 1) % _NBUF)

    return sc_gather


def kernel(grid, table, gamma, beta):
    batch, h, w = grid.shape
    v, d = table.shape
    n = batch * h * w

    tpad = jnp.zeros((16, d), jnp.float32).at[:v].set(table.astype(jnp.float32))
    nt = pl.pallas_call(
        _ln_body,
        out_shape=jax.ShapeDtypeStruct((16, d), jnp.float32),
    )(tpad, gamma.reshape(1, d), beta.reshape(1, d))

    idx = grid.reshape(n).astype(jnp.int32)
    out = _make_sc_gather(n)(nt.reshape(16 * d), idx)
    return out.reshape(batch, h * w, d)


# speedup vs baseline: 18.1393x; 11.0435x over previous
import functools

import jax
import jax.numpy as jnp
from jax import lax
from jax.experimental import pallas as pl
from jax.experimental.pallas import tpu as pltpu
from jax.experimental.pallas import tpu_sc as plsc

_EPS = 1e-5
_NC = 2
_NS = 16
_NW = _NC * _NS
_L = 16
_D = 128
_BUF = 256
_NBUF = 2


def _ln_body(t_ref, g_ref, b_ref, o_ref):
    t = t_ref[...]
    mean = jnp.mean(t, axis=1, keepdims=True)
    var = jnp.mean((t - mean) ** 2, axis=1, keepdims=True)
    o_ref[...] = (t - mean) * lax.rsqrt(var + _EPS) * g_ref[...] + b_ref[...]


def _make_sc_gather(n):
    rw = n // _NW
    steps = rw // _BUF
    groups = _BUF // _L
    mesh = plsc.VectorSubcoreMesh(core_axis_name="c", subcore_axis_name="s")

    @functools.partial(
        pl.kernel,
        out_type=jax.ShapeDtypeStruct((n * _D,), jnp.float32),
        mesh=mesh,
        compiler_params=pltpu.CompilerParams(needs_layout_passes=False),
        scratch_types=[
            pltpu.VMEM((16 * _D,), jnp.float32),
            pltpu.VMEM((rw,), jnp.int32),
            pltpu.VMEM((_BUF * _D,), jnp.float32),
            pltpu.VMEM((_BUF * _D,), jnp.float32),
            pltpu.SemaphoreType.DMA,
        ],
    )
    def sc_gather(nt_hbm, idx_hbm, out_hbm, nt_v, idx_v, buf0_v, buf1_v, ssem):
        bufs = [buf0_v, buf1_v]
        wid = lax.axis_index("s") * _NC + lax.axis_index("c")
        base = wid * rw
        pltpu.sync_copy(nt_hbm, nt_v)
        pltpu.sync_copy(idx_hbm.at[pl.ds(base, rw)], idx_v)

        lane_iota = lax.broadcasted_iota(jnp.int32, (_L,), 0)
        lane_off = lane_iota * _D

        def construct(step, b):
            @plsc.parallel_loop(0, groups * _L)
            def grp_body(i):
                g = lax.shift_right_logical(i, 4)
                j = jnp.bitwise_and(i, _L - 1)
                idx16 = idx_v[pl.ds(step * _BUF + g * _L, _L)]
                gbase = lax.shift_left(idx16, 7)
                sbase = lane_off + g * (_L * _D)
                perm = jnp.bitwise_and(lane_iota + j, _L - 1)
                for m in range(_D // _L):
                    o = perm + (_L * m)
                    v = plsc.load_gather(nt_v, [gbase + o])
                    plsc.store_scatter(bufs[b], [sbase + o], v)

        def fire_store(step, b):
            pltpu.async_copy(
                bufs[b],
                out_hbm.at[pl.ds((base + step * _BUF) * _D, _BUF * _D)],
                ssem,
            )

        def wait_store(step, b):
            pltpu.make_async_copy(
                bufs[b],
                out_hbm.at[pl.ds((base + step * _BUF) * _D, _BUF * _D)],
                ssem,
            ).wait()

        def body(go, carry):
            for b in range(_NBUF):
                step = go * _NBUF + b

                @pl.when(step >= _NBUF)
                def _():
                    wait_store(step - _NBUF, b)

                construct(step, b)
                fire_store(step, b)
            return carry

        lax.fori_loop(0, steps // _NBUF, body, 0)
        wait_store(steps - 2, (steps - 2) % _NBUF)
        wait_store(steps - 1, (steps - 1) % _NBUF)

    return sc_gather


def kernel(grid, table, gamma, beta):
    batch, h, w = grid.shape
    v, d = table.shape
    n = batch * h * w

    tpad = jnp.zeros((16, d), jnp.float32).at[:v].set(table.astype(jnp.float32))
    nt = pl.pallas_call(
        _ln_body,
        out_shape=jax.ShapeDtypeStruct((16, d), jnp.float32),
    )(tpad, gamma.reshape(1, d), beta.reshape(1, d))

    idx = grid.reshape(n).astype(jnp.int32)
    out = _make_sc_gather(n)(nt.reshape(16 * d), idx)
    return out.reshape(batch, h * w, d)
